# baseline (speedup 1.0000x reference)
import functools

import jax
import jax.numpy as jnp
from jax import lax
from jax.experimental import pallas as pl
from jax.experimental.pallas import tpu as pltpu
from jax.experimental.pallas import tpu_sc as plsc

D = 128
MAXLEN = 512
NSEG = 2
NC = 2
NS = 16
NW = NC * NS
CHUNK = 64
LANES = 16
NBUF = 4


def _sc_body(tok_hbm, pos_hbm, seg_emb_hbm, x_hbm, seg_hbm, out_hbm,
             psum_shr, xall, call, tok0, tok1, tok2, tok3,
             semt0, semt1, semt2, semt3, semo0, semo1, semo2, semo3,
             semp0, semp1, semp2, semp3):
    nchunk_w = xall.shape[0]
    w = lax.axis_index("s") * NC + lax.axis_index("c")
    iota = lax.iota(jnp.int32, LANES)
    toks = (tok0, tok1, tok2, tok3)
    semts = (semt0, semt1, semt2, semt3)
    semos = (semo0, semo1, semo2, semo3)
    semps = (semp0, semp1, semp2, semp3)

    cpx = pltpu.make_async_copy(
        x_hbm.at[pl.ds(w * nchunk_w, nchunk_w)], xall, semt1)
    cpc = pltpu.make_async_copy(
        seg_hbm.at[pl.ds(w * nchunk_w, nchunk_w)], call, semo1)
    cpx.start()
    cpc.start()

    sid = lax.axis_index("s")
    prows = (MAXLEN * NSEG) // NS
    g = sid // (MAXLEN // prows)
    s0 = lax.rem(sid * prows, MAXLEN)
    pltpu.sync_copy(seg_emb_hbm, tok0.at[pl.ds(0, NSEG)])
    pltpu.sync_copy(pos_hbm.at[pl.ds(s0, prows)], tok1.at[pl.ds(0, prows)])

    def prow_body(r, carry):
        for j in range(D // LANES):
            sl = pl.ds(j * LANES, LANES)
            tok1[r, sl] = tok1[r, sl] + tok0[g, sl]
        return carry

    lax.fori_loop(0, prows, prow_body, 0, unroll=2)
    pltpu.sync_copy(tok1.at[pl.ds(0, prows)],
                    psum_shr.at[pl.ds(sid * prows, prows)])
    plsc.subcore_barrier()

    cpx.wait()
    cpc.wait()

    def cidx_body(j, carry):
        posbase = lax.rem(j, MAXLEN // CHUNK) * CHUNK
        for i in range(CHUNK // LANES):
            sl = pl.ds(i * LANES, LANES)
            call[j, sl] = call[j, sl] * MAXLEN + (iota + (i * LANES + posbase))
        return carry

    lax.fori_loop(0, nchunk_w, cidx_body, 0)

    def fire_tok(c, b):
        pltpu.async_copy(tok_hbm.at[xall.at[c]], toks[b], semts[b])

    def wait_tok(c, b):
        pltpu.make_async_copy(tok_hbm.at[xall.at[c]], toks[b], semts[b]).wait()

    def fire_ps(c, b):
        pltpu.async_copy(psum_shr.at[call.at[c]], toks[b], semps[b], add=True)

    def wait_ps(c, b):
        pltpu.make_async_copy(psum_shr.at[call.at[c]], toks[b],
                              semps[b]).wait()

    def out_copy(c, b):
        rowbase = (w * nchunk_w + c) * CHUNK
        return pltpu.make_async_copy(
            toks[b], out_hbm.at[pl.ds(rowbase, CHUNK)], semos[b])

    def turn(c, b, start_out_prev, wait_out, fire_next):
        wait_tok(c, b)
        fire_ps(c, b)
        if start_out_prev:
            wait_ps(c - 1, (b - 1) % NBUF)
            out_copy(c - 1, (b - 1) % NBUF).start()
        if wait_out:
            out_copy(c - 2, (b - 2) % NBUF).wait()
        if fire_next:
            fire_tok(c + 2, (b + 2) % NBUF)

    fire_tok(0, 0)
    fire_tok(1, 1)
    turn(0, 0, False, False, True)
    turn(1, 1, True, False, True)

    n_uni = nchunk_w - NBUF
    assert n_uni % NBUF == 0

    def main_body(cc, carry):
        for db in range(NBUF):
            turn(NBUF * cc + 2 + db, (2 + db) % NBUF, True, True, True)
        return carry

    lax.fori_loop(0, n_uni // NBUF, main_body, 0)

    nw = nchunk_w
    turn(nw - 2, (nw - 2) % NBUF, True, True, False)
    turn(nw - 1, (nw - 1) % NBUF, True, True, False)
    wait_ps(nw - 1, (nw - 1) % NBUF)
    out_copy(nw - 1, (nw - 1) % NBUF).start()
    out_copy(nw - 2, (nw - 2) % NBUF).wait()
    out_copy(nw - 1, (nw - 1) % NBUF).wait()


def _sc_gather(tok_embed, pos_embed, seg_embed, x_blk, seg_blk):
    nblk = x_blk.shape[0]
    rows = nblk * CHUNK
    fn = functools.partial(
        pl.kernel,
        out_type=jax.ShapeDtypeStruct((rows, D), jnp.float32),
        mesh=plsc.VectorSubcoreMesh(core_axis_name="c", subcore_axis_name="s"),
        scratch_types=[
            pltpu.VMEM_SHARED((MAXLEN * NSEG, D), jnp.float32),
            pltpu.VMEM((nblk // NW, CHUNK), jnp.int32),
            pltpu.VMEM((nblk // NW, CHUNK), jnp.int32),
            pltpu.VMEM((CHUNK, D), jnp.float32),
            pltpu.VMEM((CHUNK, D), jnp.float32),
            pltpu.VMEM((CHUNK, D), jnp.float32),
            pltpu.VMEM((CHUNK, D), jnp.float32),
            pltpu.SemaphoreType.DMA,
            pltpu.SemaphoreType.DMA,
            pltpu.SemaphoreType.DMA,
            pltpu.SemaphoreType.DMA,
            pltpu.SemaphoreType.DMA,
            pltpu.SemaphoreType.DMA,
            pltpu.SemaphoreType.DMA,
            pltpu.SemaphoreType.DMA,
            pltpu.SemaphoreType.DMA,
            pltpu.SemaphoreType.DMA,
            pltpu.SemaphoreType.DMA,
            pltpu.SemaphoreType.DMA,
        ],
    )(_sc_body)
    return fn(tok_embed, pos_embed, seg_embed, x_blk, seg_blk)


def kernel(x, seg, tok_embed, pos_embed, seg_embed):
    batch, seqlen = x.shape
    x_blk = x.reshape(-1, CHUNK).astype(jnp.int32)
    seg_blk = seg.reshape(-1, CHUNK).astype(jnp.int32)
    out = _sc_gather(tok_embed, pos_embed, seg_embed, x_blk, seg_blk)
    return out.reshape(batch, seqlen, D)


# speedup vs baseline: 1.0196x; 1.0196x over previous
import functools

import jax
import jax.numpy as jnp
from jax import lax
from jax.experimental import pallas as pl
from jax.experimental.pallas import tpu as pltpu
from jax.experimental.pallas import tpu_sc as plsc

D = 128
MAXLEN = 512
NSEG = 2
NC = 2
NS = 16
NW = NC * NS
CHUNK = 64
LANES = 16
NBUF = 3


def _sc_body(tok_hbm, pos_hbm, seg_emb_hbm, x_hbm, seg_hbm, out_hbm,
             psum_shr, xall, call, tok0, ps0, tok1, ps1, tok2, ps2,
             semt0, semp0, semo0, semt1, semp1, semo1, semt2, semp2, semo2):
    nchunk_w = xall.shape[0]
    w = lax.axis_index("s") * NC + lax.axis_index("c")
    iota = lax.iota(jnp.int32, LANES)
    toks = (tok0, tok1, tok2)
    pss = (ps0, ps1, ps2)
    semts = (semt0, semt1, semt2)
    semps = (semp0, semp1, semp2)
    semos = (semo0, semo1, semo2)

    cpx = pltpu.make_async_copy(
        x_hbm.at[pl.ds(w * xall.shape[0], xall.shape[0])], xall, semt1)
    cpc = pltpu.make_async_copy(
        seg_hbm.at[pl.ds(w * call.shape[0], call.shape[0])], call, semp1)
    cpx.start()
    cpc.start()

    sid = lax.axis_index("s")
    prows = (MAXLEN * NSEG) // NS
    g = sid // (MAXLEN // prows)
    s0 = lax.rem(sid * prows, MAXLEN)
    pltpu.sync_copy(seg_emb_hbm, tok0.at[pl.ds(0, NSEG)])
    pltpu.sync_copy(pos_hbm.at[pl.ds(s0, prows)], ps0.at[pl.ds(0, prows)])

    def prow_body(r, carry):
        for j in range(D // LANES):
            sl = pl.ds(j * LANES, LANES)
            ps0[r, sl] = ps0[r, sl] + tok0[g, sl]
        return carry

    lax.fori_loop(0, prows, prow_body, 0, unroll=2)
    pltpu.sync_copy(ps0.at[pl.ds(0, prows)],
                    psum_shr.at[pl.ds(sid * prows, prows)])
    plsc.subcore_barrier()

    cpx.wait()
    cpc.wait()

    def cidx_body(j, carry):
        posbase = lax.rem(j, MAXLEN // CHUNK) * CHUNK
        for i in range(CHUNK // LANES):
            sl = pl.ds(i * LANES, LANES)
            call[j, sl] = call[j, sl] * MAXLEN + (iota + (i * LANES + posbase))
        return carry

    lax.fori_loop(0, nchunk_w, cidx_body, 0)

    def fire_tok(c, b):
        pltpu.async_copy(tok_hbm.at[xall.at[c]], toks[b], semts[b])

    def fire_ps(c, b):
        pltpu.async_copy(psum_shr.at[call.at[c]], pss[b], semps[b])

    def out_copy(c, b):
        rowbase = (w * nchunk_w + c) * CHUNK
        return pltpu.make_async_copy(
            pss[b], out_hbm.at[pl.ds(rowbase, CHUNK)], semos[b])

    def turn(c, b, bprev, waitprev, refire_tok, refire_ps):
        pltpu.make_async_copy(tok_hbm.at[xall.at[c]], toks[b], semts[b]).wait()
        pltpu.make_async_copy(psum_shr.at[call.at[c]], pss[b], semps[b]).wait()

        @plsc.parallel_loop(0, CHUNK, unroll=8)
        def add_body(r):
            for j in range(D // LANES):
                sl = pl.ds(j * LANES, LANES)
                plsc.addupdate(pss[b].at[r, sl], toks[b][r, sl])

        out_copy(c, b).start()
        if refire_tok:
            fire_tok(c + NBUF, b)
        if waitprev:
            out_copy(c - 1, bprev).wait()
            if refire_ps:
                fire_ps(c + NBUF - 1, bprev)

    for c in range(NBUF):
        fire_tok(c, c)
        fire_ps(c, c)

    turn(0, 0, NBUF - 1, False, True, False)

    n_uni = nchunk_w - 1 - NBUF
    assert n_uni % NBUF == 0

    def main_body(cc, carry):
        for db in range(NBUF):
            c = NBUF * cc + 1 + db
            turn(c, (1 + db) % NBUF, db, True, True, True)
        return carry

    lax.fori_loop(0, n_uni // NBUF, main_body, 0)

    c0 = nchunk_w - NBUF
    turn(c0, c0 % NBUF, (c0 - 1) % NBUF, True, False, True)
    turn(c0 + 1, (c0 + 1) % NBUF, c0 % NBUF, True, False, False)
    turn(c0 + 2, (c0 + 2) % NBUF, (c0 + 1) % NBUF, True, False, False)
    out_copy(nchunk_w - 1, (nchunk_w - 1) % NBUF).wait()


def _sc_gather(tok_embed, pos_embed, seg_embed, x_blk, seg_blk):
    nblk = x_blk.shape[0]
    rows = nblk * CHUNK
    fn = functools.partial(
        pl.kernel,
        out_type=jax.ShapeDtypeStruct((rows, D), jnp.float32),
        mesh=plsc.VectorSubcoreMesh(core_axis_name="c", subcore_axis_name="s"),
        scratch_types=[
            pltpu.VMEM_SHARED((MAXLEN * NSEG, D), jnp.float32),
            pltpu.VMEM((nblk // NW, CHUNK), jnp.int32),
            pltpu.VMEM((nblk // NW, CHUNK), jnp.int32),
            pltpu.VMEM((CHUNK, D), jnp.float32),
            pltpu.VMEM((CHUNK, D), jnp.float32),
            pltpu.VMEM((CHUNK, D), jnp.float32),
            pltpu.VMEM((CHUNK, D), jnp.float32),
            pltpu.VMEM((CHUNK, D), jnp.float32),
            pltpu.VMEM((CHUNK, D), jnp.float32),
            pltpu.SemaphoreType.DMA,
            pltpu.SemaphoreType.DMA,
            pltpu.SemaphoreType.DMA,
            pltpu.SemaphoreType.DMA,
            pltpu.SemaphoreType.DMA,
            pltpu.SemaphoreType.DMA,
            pltpu.SemaphoreType.DMA,
            pltpu.SemaphoreType.DMA,
            pltpu.SemaphoreType.DMA,
        ],
    )(_sc_body)
    return fn(tok_embed, pos_embed, seg_embed, x_blk, seg_blk)


def kernel(x, seg, tok_embed, pos_embed, seg_embed):
    batch, seqlen = x.shape
    x_blk = x.reshape(-1, CHUNK).astype(jnp.int32)
    seg_blk = seg.reshape(-1, CHUNK).astype(jnp.int32)
    out = _sc_gather(tok_embed, pos_embed, seg_embed, x_blk, seg_blk)
    return out.reshape(batch, seqlen, D)
